# baseline (speedup 1.0000x reference)
import jax
import jax.numpy as jnp
from jax import lax
from jax.experimental import pallas as pl
from jax.experimental.pallas import tpu as pltpu
from jax.experimental.pallas import tpu_sc as plsc

L = 16
R, C = 64, 32768
K = 3
NC, NS = 2, 16
NW = NC * NS
RSC = NW
RTC = R - RSC
C2 = C // 2
ACC = 4
TCB = 8192

_NEG = float("-inf")


def _insert(t0, t1, t2, v):
    lo = jnp.minimum(t0, v)
    t0 = jnp.maximum(t0, v)
    lo2 = jnp.minimum(t1, lo)
    t1 = jnp.maximum(t1, lo)
    t2 = jnp.maximum(t2, lo2)
    return t0, t1, t2


def _sc_body(x_hbm, out_hbm, xv, resv, s0, s1, s2, s3):
    cid = lax.axis_index("c")
    sid = lax.axis_index("s")
    wid = sid * NC + cid
    sems = [s0, s1, s2, s3]
    nseg = len(sems)
    seg = C // nseg
    copies = [
        pltpu.async_copy(x_hbm.at[wid, pl.ds(h * seg, seg)],
                         xv.at[pl.ds(h * seg, seg)], sems[h])
        for h in range(nseg)]
    lane = lax.iota(jnp.int32, L)
    full = jnp.full((L,), _NEG, jnp.float32)
    acc = [(full, full, full)] * ACC
    for h in range(nseg):
        copies[h].wait()

        def step(i, carry, _h=h):
            acc = list(carry)
            for a in range(ACC):
                off = _h * seg + (i * ACC + a) * L
                acc[a] = _insert(*acc[a], xv[pl.ds(off, L)])
            return tuple(acc)

        acc = list(lax.fori_loop(0, seg // (L * ACC), step, tuple(acc)))

    t0, t1, t2 = acc[0]
    for a in range(1, ACC):
        for v in acc[a]:
            t0, t1, t2 = _insert(t0, t1, t2, v)

    def pop(t0, t1, t2):
        m = jnp.max(t0)
        j = plsc.all_reduce_ffs(t0 == m)
        sel = lane == j
        return (m, jnp.where(sel, t1, t0), jnp.where(sel, t2, t1),
                jnp.where(sel, _NEG, t2))

    m1, t0, t1, t2 = pop(t0, t1, t2)
    m2, t0, t1, t2 = pop(t0, t1, t2)
    m3 = jnp.max(t0)
    res = jnp.where(lane == 0, m1,
                    jnp.where(lane == 1, m2,
                              jnp.where(lane == 2, m3, jnp.float32(0.0))))
    resv[...] = res
    pltpu.sync_copy(resv.at[pl.ds(0, 8)], out_hbm.at[pl.ds(wid * 128, 8)])


def _tc_body(x_ref, o_ref, t0, t1, t2):
    c = pl.program_id(0)

    @pl.when(c == 0)
    def _init():
        t0[...] = jnp.full((RTC, 128), _NEG, jnp.float32)
        t1[...] = jnp.full((RTC, 128), _NEG, jnp.float32)
        t2[...] = jnp.full((RTC, 128), _NEG, jnp.float32)

    blk = x_ref[...]
    a0, a1, a2 = t0[...], t1[...], t2[...]
    for j in range(TCB // 128):
        a0, a1, a2 = _insert(a0, a1, a2, blk[:, 128 * j:128 * (j + 1)])
    t0[...], t1[...], t2[...] = a0, a1, a2

    @pl.when(c == C // TCB - 1)
    def _fin():
        a0, a1, a2 = t0[...], t1[...], t2[...]
        iota = lax.broadcasted_iota(jnp.int32, (RTC, 128), 1)

        def pop(a0, a1, a2):
            m = jnp.max(a0, axis=1, keepdims=True)
            j = jnp.min(jnp.where(a0 == m, iota, 128), axis=1, keepdims=True)
            sel = iota == j
            return (m, jnp.where(sel, a1, a0), jnp.where(sel, a2, a1),
                    jnp.where(sel, _NEG, a2))

        m1, a0, a1, a2 = pop(a0, a1, a2)
        m2, a0, a1, a2 = pop(a0, a1, a2)
        m3 = jnp.max(a0, axis=1, keepdims=True)
        res = jnp.where(iota == 0, m1,
                        jnp.where(iota == 1, m2,
                                  jnp.where(iota == 2, m3, jnp.float32(0.0))))
        o_ref[...] = res[:, :8]


def _asm_body(sc_ref, tc_ref, o_ref):
    o_ref[0:RSC, :] = sc_ref[...][:, :K]
    o_ref[RSC:R, :] = tc_ref[...][:, :K]


def kernel(x):
    mesh = plsc.VectorSubcoreMesh(core_axis_name="c", subcore_axis_name="s")
    f_sc = pl.kernel(
        _sc_body,
        mesh=mesh,
        compiler_params=pltpu.CompilerParams(needs_layout_passes=False),
        out_type=jax.ShapeDtypeStruct((RSC * 128,), jnp.float32),
        scratch_types=[
            pltpu.VMEM((C,), jnp.float32),
            pltpu.VMEM((L,), jnp.float32),
            pltpu.SemaphoreType.DMA,
            pltpu.SemaphoreType.DMA,
            pltpu.SemaphoreType.DMA,
            pltpu.SemaphoreType.DMA,
        ],
    )
    f_tc = pl.pallas_call(
        _tc_body,
        grid=(C // TCB,),
        in_specs=[pl.BlockSpec((RTC, TCB), lambda c: (1, c))],
        out_specs=pl.BlockSpec((RTC, 8), lambda c: (0, 0)),
        out_shape=jax.ShapeDtypeStruct((RTC, 8), jnp.float32),
        scratch_shapes=[pltpu.VMEM((RTC, 128), jnp.float32)] * 3,
    )
    f_asm = pl.pallas_call(
        _asm_body,
        out_shape=jax.ShapeDtypeStruct((R, K), jnp.float32),
    )
    return f_asm(f_sc(x).reshape(RSC, 128), f_tc(x))


# speedup vs baseline: 1.0590x; 1.0590x over previous
import jax
import jax.numpy as jnp
from jax import lax
from jax.experimental import pallas as pl
from jax.experimental.pallas import tpu as pltpu
from jax.experimental.pallas import tpu_sc as plsc

L = 16
R, C = 64, 32768
K = 3
NC, NS = 2, 16
NW = NC * NS
RSC = NW
RTC = R - RSC
C2 = C // 2
ACC = 4
UNR = 8
TCB = 8192

_NEG = float("-inf")


def _insert(t0, t1, t2, v):
    lo = jnp.minimum(t0, v)
    t0 = jnp.maximum(t0, v)
    lo2 = jnp.minimum(t1, lo)
    t1 = jnp.maximum(t1, lo)
    t2 = jnp.maximum(t2, lo2)
    return t0, t1, t2


def _sc_body(x_hbm, out_hbm, xv, resv, s0, s1, s2, s3):
    cid = lax.axis_index("c")
    sid = lax.axis_index("s")
    wid = sid * NC + cid
    sems = [s0, s1, s2, s3]
    nseg = len(sems)
    seg = C // nseg
    copies = [
        pltpu.async_copy(x_hbm.at[wid, pl.ds(h * seg, seg)],
                         xv.at[pl.ds(h * seg, seg)], sems[h])
        for h in range(nseg)]
    lane = lax.iota(jnp.int32, L)
    full = jnp.full((L,), _NEG, jnp.float32)
    acc = [(full, full, full)] * ACC
    for h in range(nseg):
        copies[h].wait()

        def step(i, carry, _h=h):
            acc = list(carry)
            for j in range(UNR):
                a = j % ACC
                off = _h * seg + (i * UNR + j) * L
                acc[a] = _insert(*acc[a], xv[pl.ds(off, L)])
            return tuple(acc)

        acc = list(lax.fori_loop(0, seg // (L * UNR), step, tuple(acc)))

    t0, t1, t2 = acc[0]
    for a in range(1, ACC):
        for v in acc[a]:
            t0, t1, t2 = _insert(t0, t1, t2, v)

    def pop(t0, t1, t2):
        m = jnp.max(t0)
        j = plsc.all_reduce_ffs(t0 == m)
        sel = lane == j
        return (m, jnp.where(sel, t1, t0), jnp.where(sel, t2, t1),
                jnp.where(sel, _NEG, t2))

    m1, t0, t1, t2 = pop(t0, t1, t2)
    m2, t0, t1, t2 = pop(t0, t1, t2)
    m3 = jnp.max(t0)
    res = jnp.where(lane == 0, m1,
                    jnp.where(lane == 1, m2,
                              jnp.where(lane == 2, m3, jnp.float32(0.0))))
    resv[...] = res
    pltpu.sync_copy(resv.at[pl.ds(0, 8)], out_hbm.at[pl.ds(wid * 8, 8)])


def _tc_body(x_ref, o_ref, t0, t1, t2):
    c = pl.program_id(0)

    @pl.when(c == 0)
    def _init():
        t0[...] = jnp.full((RTC, 128), _NEG, jnp.float32)
        t1[...] = jnp.full((RTC, 128), _NEG, jnp.float32)
        t2[...] = jnp.full((RTC, 128), _NEG, jnp.float32)

    blk = x_ref[...]
    a0, a1, a2 = t0[...], t1[...], t2[...]
    for j in range(TCB // 128):
        a0, a1, a2 = _insert(a0, a1, a2, blk[:, 128 * j:128 * (j + 1)])
    t0[...], t1[...], t2[...] = a0, a1, a2

    @pl.when(c == C // TCB - 1)
    def _fin():
        a0, a1, a2 = t0[...], t1[...], t2[...]
        iota = lax.broadcasted_iota(jnp.int32, (RTC, 128), 1)

        def pop(a0, a1, a2):
            m = jnp.max(a0, axis=1, keepdims=True)
            j = jnp.min(jnp.where(a0 == m, iota, 128), axis=1, keepdims=True)
            sel = iota == j
            return (m, jnp.where(sel, a1, a0), jnp.where(sel, a2, a1),
                    jnp.where(sel, _NEG, a2))

        m1, a0, a1, a2 = pop(a0, a1, a2)
        m2, a0, a1, a2 = pop(a0, a1, a2)
        m3 = jnp.max(a0, axis=1, keepdims=True)
        res = jnp.where(iota == 0, m1,
                        jnp.where(iota == 1, m2,
                                  jnp.where(iota == 2, m3, jnp.float32(0.0))))
        o_ref[...] = res[:, :8]


def kernel(x):
    mesh = plsc.VectorSubcoreMesh(core_axis_name="c", subcore_axis_name="s")
    f_sc = pl.kernel(
        _sc_body,
        mesh=mesh,
        compiler_params=pltpu.CompilerParams(needs_layout_passes=False),
        out_type=jax.ShapeDtypeStruct((RSC * 8,), jnp.float32),
        scratch_types=[
            pltpu.VMEM((C,), jnp.float32),
            pltpu.VMEM((L,), jnp.float32),
            pltpu.SemaphoreType.DMA,
            pltpu.SemaphoreType.DMA,
            pltpu.SemaphoreType.DMA,
            pltpu.SemaphoreType.DMA,
        ],
    )
    f_tc = pl.pallas_call(
        _tc_body,
        grid=(C // TCB,),
        in_specs=[pl.BlockSpec((RTC, TCB), lambda c: (1, c))],
        out_specs=pl.BlockSpec((RTC, 8), lambda c: (0, 0)),
        out_shape=jax.ShapeDtypeStruct((RTC, 8), jnp.float32),
        scratch_shapes=[pltpu.VMEM((RTC, 128), jnp.float32)] * 3,
    )
    out8 = jnp.concatenate([f_sc(x).reshape(RSC, 8), f_tc(x)], axis=0)
    return out8[:, :K]


# speedup vs baseline: 1.0593x; 1.0003x over previous
import jax
import jax.numpy as jnp
from jax import lax
from jax.experimental import pallas as pl
from jax.experimental.pallas import tpu as pltpu
from jax.experimental.pallas import tpu_sc as plsc

L = 16
R, C = 64, 32768
K = 3
NC, NS = 2, 16
NW = NC * NS
RSC = NW
RTC = R - RSC
C2 = C // 2
ACC = 4
UNR = 8
TCB = 8192

_NEG = float("-inf")


def _insert(t0, t1, t2, v):
    lo = jnp.minimum(t0, v)
    t0 = jnp.maximum(t0, v)
    lo2 = jnp.minimum(t1, lo)
    t1 = jnp.maximum(t1, lo)
    t2 = jnp.maximum(t2, lo2)
    return t0, t1, t2


def _sc_body(x_hbm, out_hbm, xv, resv, s0, s1, s2, s3):
    cid = lax.axis_index("c")
    sid = lax.axis_index("s")
    wid = sid * NC + cid
    sems = [s0, s1, s2, s3]
    nseg = len(sems)
    seg = C // nseg
    copies = [
        pltpu.async_copy(x_hbm.at[wid, pl.ds(h * seg, seg)],
                         xv.at[pl.ds(h * seg, seg)], sems[h])
        for h in range(nseg)]
    lane = lax.iota(jnp.int32, L)
    full = jnp.full((L,), _NEG, jnp.float32)
    acc = [(full, full, full)] * ACC
    for h in range(nseg):
        copies[h].wait()

        def step(i, carry, _h=h):
            acc = list(carry)
            for j in range(UNR):
                a = j % ACC
                off = _h * seg + (i * UNR + j) * L
                acc[a] = _insert(*acc[a], xv[pl.ds(off, L)])
            return tuple(acc)

        acc = list(lax.fori_loop(0, seg // (L * UNR), step, tuple(acc)))

    t0, t1, t2 = acc[0]
    for a in range(1, ACC):
        for v in acc[a]:
            t0, t1, t2 = _insert(t0, t1, t2, v)

    def pop(t0, t1, t2):
        m = jnp.max(t0)
        j = plsc.all_reduce_ffs(t0 == m)
        sel = lane == j
        return (m, jnp.where(sel, t1, t0), jnp.where(sel, t2, t1),
                jnp.where(sel, _NEG, t2))

    m1, t0, t1, t2 = pop(t0, t1, t2)
    m2, t0, t1, t2 = pop(t0, t1, t2)
    m3 = jnp.max(t0)
    res = jnp.where(lane == 0, m1,
                    jnp.where(lane == 1, m2,
                              jnp.where(lane == 2, m3, jnp.float32(0.0))))
    resv[...] = res
    pltpu.sync_copy(resv.at[pl.ds(0, 8)], out_hbm.at[pl.ds(wid * 8, 8)])


def _tc_body(x_ref, o_ref, t0, t1, t2):
    c = pl.program_id(0)

    @pl.when(c == 0)
    def _init():
        t0[...] = jnp.full((RTC, 128), _NEG, jnp.float32)
        t1[...] = jnp.full((RTC, 128), _NEG, jnp.float32)
        t2[...] = jnp.full((RTC, 128), _NEG, jnp.float32)

    blk = x_ref[...]
    a0, a1, a2 = t0[...], t1[...], t2[...]
    for j in range(TCB // 128):
        a0, a1, a2 = _insert(a0, a1, a2, blk[:, 128 * j:128 * (j + 1)])
    t0[...], t1[...], t2[...] = a0, a1, a2

    @pl.when(c == C // TCB - 1)
    def _fin():
        a0, a1, a2 = t0[...], t1[...], t2[...]
        iota = lax.broadcasted_iota(jnp.int32, (RTC, 128), 1)

        def pop(a0, a1, a2):
            m = jnp.max(a0, axis=1, keepdims=True)
            j = jnp.min(jnp.where(a0 == m, iota, 128), axis=1, keepdims=True)
            sel = iota == j
            return (m, jnp.where(sel, a1, a0), jnp.where(sel, a2, a1),
                    jnp.where(sel, _NEG, a2))

        m1, a0, a1, a2 = pop(a0, a1, a2)
        m2, a0, a1, a2 = pop(a0, a1, a2)
        m3 = jnp.max(a0, axis=1, keepdims=True)
        res = jnp.where(iota == 0, m1,
                        jnp.where(iota == 1, m2,
                                  jnp.where(iota == 2, m3, jnp.float32(0.0))))
        o_ref[...] = res[:, :8]


def kernel(x):
    mesh = plsc.VectorSubcoreMesh(core_axis_name="c", subcore_axis_name="s")
    f_sc = pl.kernel(
        _sc_body,
        mesh=mesh,
        compiler_params=pltpu.CompilerParams(needs_layout_passes=False,
                                             skip_device_barrier=True),
        out_type=jax.ShapeDtypeStruct((RSC * 8,), jnp.float32),
        scratch_types=[
            pltpu.VMEM((C,), jnp.float32),
            pltpu.VMEM((L,), jnp.float32),
            pltpu.SemaphoreType.DMA,
            pltpu.SemaphoreType.DMA,
            pltpu.SemaphoreType.DMA,
            pltpu.SemaphoreType.DMA,
        ],
    )
    f_tc = pl.pallas_call(
        _tc_body,
        grid=(C // TCB,),
        in_specs=[pl.BlockSpec((RTC, TCB), lambda c: (1, c))],
        out_specs=pl.BlockSpec((RTC, 8), lambda c: (0, 0)),
        out_shape=jax.ShapeDtypeStruct((RTC, 8), jnp.float32),
        scratch_shapes=[pltpu.VMEM((RTC, 128), jnp.float32)] * 3,
    )
    out8 = jnp.concatenate([f_sc(x).reshape(RSC, 8), f_tc(x)], axis=0)
    return out8[:, :K]


# speedup vs baseline: 1.0635x; 1.0040x over previous
import jax
import jax.numpy as jnp
from jax import lax
from jax.experimental import pallas as pl
from jax.experimental.pallas import tpu as pltpu
from jax.experimental.pallas import tpu_sc as plsc

L = 16
R, C = 64, 32768
K = 3
NC, NS = 2, 16
NW = NC * NS
RSC = NW
RTC = R - RSC
C2 = C // 2
ACC = 4
UNR = 8
TCB = 8192

_NEG = float("-inf")


def _insert(t0, t1, t2, v):
    lo = jnp.minimum(t0, v)
    t0 = jnp.maximum(t0, v)
    lo2 = jnp.minimum(t1, lo)
    t1 = jnp.maximum(t1, lo)
    t2 = jnp.maximum(t2, lo2)
    return t0, t1, t2


def _sc_body(x_hbm, out_hbm, xv, resv, s0, s1, s2, s3):
    cid = lax.axis_index("c")
    sid = lax.axis_index("s")
    wid = sid * NC + cid
    sems = [s0, s1, s2, s3]
    nseg = len(sems)
    seg = C // nseg
    copies = [
        pltpu.async_copy(x_hbm.at[wid, pl.ds(h * seg, seg)],
                         xv.at[pl.ds(h * seg, seg)], sems[h])
        for h in range(nseg)]
    lane = lax.iota(jnp.int32, L)
    full = jnp.full((L,), _NEG, jnp.float32)
    acc = [(full, full, full)] * ACC
    for h in range(nseg):
        copies[h].wait()

        @plsc.parallel_loop(0, seg // (L * ACC), 1, unroll=2,
                            carry=tuple(acc))
        def _loop(i, carry, _h=h):
            acc = list(carry)
            for a in range(ACC):
                off = _h * seg + (i * ACC + a) * L
                acc[a] = _insert(*acc[a], xv[pl.ds(off, L)])
            return tuple(acc)

        acc = list(_loop)

    t0, t1, t2 = acc[0]
    for a in range(1, ACC):
        for v in acc[a]:
            t0, t1, t2 = _insert(t0, t1, t2, v)

    def pop(t0, t1, t2):
        m = jnp.max(t0)
        j = plsc.all_reduce_ffs(t0 == m)
        sel = lane == j
        return (m, jnp.where(sel, t1, t0), jnp.where(sel, t2, t1),
                jnp.where(sel, _NEG, t2))

    m1, t0, t1, t2 = pop(t0, t1, t2)
    m2, t0, t1, t2 = pop(t0, t1, t2)
    m3 = jnp.max(t0)
    res = jnp.where(lane == 0, m1,
                    jnp.where(lane == 1, m2,
                              jnp.where(lane == 2, m3, jnp.float32(0.0))))
    resv[...] = res
    pltpu.sync_copy(resv.at[pl.ds(0, 8)], out_hbm.at[pl.ds(wid * 8, 8)])


def _tc_body(x_ref, o_ref, t0, t1, t2):
    c = pl.program_id(0)

    @pl.when(c == 0)
    def _init():
        t0[...] = jnp.full((RTC, 128), _NEG, jnp.float32)
        t1[...] = jnp.full((RTC, 128), _NEG, jnp.float32)
        t2[...] = jnp.full((RTC, 128), _NEG, jnp.float32)

    blk = x_ref[...]
    a0, a1, a2 = t0[...], t1[...], t2[...]
    for j in range(TCB // 128):
        a0, a1, a2 = _insert(a0, a1, a2, blk[:, 128 * j:128 * (j + 1)])
    t0[...], t1[...], t2[...] = a0, a1, a2

    @pl.when(c == C // TCB - 1)
    def _fin():
        a0, a1, a2 = t0[...], t1[...], t2[...]
        iota = lax.broadcasted_iota(jnp.int32, (RTC, 128), 1)

        def pop(a0, a1, a2):
            m = jnp.max(a0, axis=1, keepdims=True)
            j = jnp.min(jnp.where(a0 == m, iota, 128), axis=1, keepdims=True)
            sel = iota == j
            return (m, jnp.where(sel, a1, a0), jnp.where(sel, a2, a1),
                    jnp.where(sel, _NEG, a2))

        m1, a0, a1, a2 = pop(a0, a1, a2)
        m2, a0, a1, a2 = pop(a0, a1, a2)
        m3 = jnp.max(a0, axis=1, keepdims=True)
        res = jnp.where(iota == 0, m1,
                        jnp.where(iota == 1, m2,
                                  jnp.where(iota == 2, m3, jnp.float32(0.0))))
        o_ref[...] = res[:, :8]


def kernel(x):
    mesh = plsc.VectorSubcoreMesh(core_axis_name="c", subcore_axis_name="s")
    f_sc = pl.kernel(
        _sc_body,
        mesh=mesh,
        compiler_params=pltpu.CompilerParams(needs_layout_passes=False),
        out_type=jax.ShapeDtypeStruct((RSC * 8,), jnp.float32),
        scratch_types=[
            pltpu.VMEM((C,), jnp.float32),
            pltpu.VMEM((L,), jnp.float32),
            pltpu.SemaphoreType.DMA,
            pltpu.SemaphoreType.DMA,
            pltpu.SemaphoreType.DMA,
            pltpu.SemaphoreType.DMA,
        ],
    )
    f_tc = pl.pallas_call(
        _tc_body,
        grid=(C // TCB,),
        in_specs=[pl.BlockSpec((RTC, TCB), lambda c: (1, c))],
        out_specs=pl.BlockSpec((RTC, 8), lambda c: (0, 0)),
        out_shape=jax.ShapeDtypeStruct((RTC, 8), jnp.float32),
        scratch_shapes=[pltpu.VMEM((RTC, 128), jnp.float32)] * 3,
    )
    out8 = jnp.concatenate([f_sc(x).reshape(RSC, 8), f_tc(x)], axis=0)
    return out8[:, :K]


# speedup vs baseline: 1.0740x; 1.0099x over previous
import jax
import jax.numpy as jnp
from jax import lax
from jax.experimental import pallas as pl
from jax.experimental.pallas import tpu as pltpu
from jax.experimental.pallas import tpu_sc as plsc

L = 16
R, C = 64, 32768
K = 3
NC, NS = 2, 16
NW = NC * NS
RSC = NW
RTC = R - RSC
C2 = C // 2
ACC = 4
TCB = 8192

_NEG = float("-inf")


def _insert(t0, t1, t2, v):
    lo = jnp.minimum(t0, v)
    t0 = jnp.maximum(t0, v)
    lo2 = jnp.minimum(t1, lo)
    t1 = jnp.maximum(t1, lo)
    t2 = jnp.maximum(t2, lo2)
    return t0, t1, t2


def _sc_body(x_hbm, out_hbm, xv, resv, s0, s1, s2, s3):
    cid = lax.axis_index("c")
    sid = lax.axis_index("s")
    wid = sid * NC + cid
    sems = [s0, s1, s2, s3]
    nseg = len(sems)
    seg = C // nseg
    copies = [
        pltpu.async_copy(x_hbm.at[wid, pl.ds(h * seg, seg)],
                         xv.at[pl.ds(h * seg, seg)], sems[h])
        for h in range(nseg)]
    lane = lax.iota(jnp.int32, L)
    full = jnp.full((L,), _NEG, jnp.float32)
    acc = [(full, full, full)] * ACC
    for h in range(nseg):
        copies[h].wait()

        def step(i, carry, _h=h):
            acc = list(carry)
            for a in range(ACC):
                off = _h * seg + (i * ACC + a) * L
                acc[a] = _insert(*acc[a], xv[pl.ds(off, L)])
            return tuple(acc)

        acc = list(lax.fori_loop(0, seg // (L * ACC), step, tuple(acc)))

    t0, t1, t2 = acc[0]
    for a in range(1, ACC):
        for v in acc[a]:
            t0, t1, t2 = _insert(t0, t1, t2, v)

    def pop(t0, t1, t2):
        m = jnp.max(t0)
        j = plsc.all_reduce_ffs(t0 == m)
        sel = lane == j
        return (m, jnp.where(sel, t1, t0), jnp.where(sel, t2, t1),
                jnp.where(sel, _NEG, t2))

    m1, t0, t1, t2 = pop(t0, t1, t2)
    m2, t0, t1, t2 = pop(t0, t1, t2)
    m3 = jnp.max(t0)
    res = jnp.where(lane == 0, m1,
                    jnp.where(lane == 1, m2,
                              jnp.where(lane == 2, m3, jnp.float32(0.0))))
    resv[...] = res
    pltpu.sync_copy(resv.at[pl.ds(0, 8)], out_hbm.at[pl.ds(wid * 8, 8)])


def _tc_body(x_ref, o_ref, t0, t1, t2):
    c = pl.program_id(0)

    @pl.when(c == 0)
    def _init():
        t0[...] = jnp.full((RTC, 128), _NEG, jnp.float32)
        t1[...] = jnp.full((RTC, 128), _NEG, jnp.float32)
        t2[...] = jnp.full((RTC, 128), _NEG, jnp.float32)

    blk = x_ref[...]
    a0, a1, a2 = t0[...], t1[...], t2[...]
    for j in range(TCB // 128):
        a0, a1, a2 = _insert(a0, a1, a2, blk[:, 128 * j:128 * (j + 1)])
    t0[...], t1[...], t2[...] = a0, a1, a2

    @pl.when(c == C // TCB - 1)
    def _fin():
        a0, a1, a2 = t0[...], t1[...], t2[...]
        iota = lax.broadcasted_iota(jnp.int32, (RTC, 128), 1)

        def pop(a0, a1, a2):
            m = jnp.max(a0, axis=1, keepdims=True)
            j = jnp.min(jnp.where(a0 == m, iota, 128), axis=1, keepdims=True)
            sel = iota == j
            return (m, jnp.where(sel, a1, a0), jnp.where(sel, a2, a1),
                    jnp.where(sel, _NEG, a2))

        m1, a0, a1, a2 = pop(a0, a1, a2)
        m2, a0, a1, a2 = pop(a0, a1, a2)
        m3 = jnp.max(a0, axis=1, keepdims=True)
        res = jnp.where(iota == 0, m1,
                        jnp.where(iota == 1, m2,
                                  jnp.where(iota == 2, m3, jnp.float32(0.0))))
        o_ref[...] = res[:, :8]


def kernel(x):
    mesh = plsc.VectorSubcoreMesh(core_axis_name="c", subcore_axis_name="s")
    f_sc = pl.kernel(
        _sc_body,
        mesh=mesh,
        compiler_params=pltpu.CompilerParams(needs_layout_passes=False),
        out_type=jax.ShapeDtypeStruct((RSC * 8,), jnp.float32),
        scratch_types=[
            pltpu.VMEM((C,), jnp.float32),
            pltpu.VMEM((L,), jnp.float32),
            pltpu.SemaphoreType.DMA,
            pltpu.SemaphoreType.DMA,
            pltpu.SemaphoreType.DMA,
            pltpu.SemaphoreType.DMA,
        ],
    )
    f_tc = pl.pallas_call(
        _tc_body,
        grid=(C // TCB,),
        in_specs=[pl.BlockSpec((RTC, TCB), lambda c: (1, c))],
        out_specs=pl.BlockSpec((RTC, 8), lambda c: (0, 0)),
        out_shape=jax.ShapeDtypeStruct((RTC, 8), jnp.float32),
        scratch_shapes=[pltpu.VMEM((RTC, 128), jnp.float32)] * 3,
    )
    out8 = jnp.concatenate([f_sc(x).reshape(RSC, 8), f_tc(x)], axis=0)
    return out8[:, :K]
